# baseline (speedup 1.0000x reference)
import functools

import jax
import jax.numpy as jnp
from jax import lax
from jax.experimental import pallas as pl
from jax.experimental.pallas import tpu as pltpu
from jax.experimental.pallas import tpu_sc as plsc

BATCH = 4096
HIST = 200
DIM = 64
TOTAL = BATCH * HIST
NUM_EMB = 1000000

_INFO = plsc.get_sparse_core_info()
NC = _INFO.num_cores
NS = _INFO.num_subcores
NW = NC * NS
PER_W = TOTAL // NW
CHUNK = 512
NCHUNK = PER_W // CHUNK
NPAIR = NCHUNK // 2

TBLK = 2048
BBLK = 128


def _tc_table_transpose(wt):
    grid = (NUM_EMB + TBLK - 1) // TBLK

    def body(x_ref, o_ref):
        o_ref[...] = x_ref[...].T

    return pl.pallas_call(
        body,
        grid=(grid,),
        in_specs=[pl.BlockSpec((DIM, TBLK), lambda k: (0, k))],
        out_specs=pl.BlockSpec((TBLK, DIM), lambda k: (k, 0)),
        out_shape=jax.ShapeDtypeStruct((NUM_EMB, DIM), jnp.float32),
    )(wt)


def _tc_out_convert(arr3):
    grid = BATCH // BBLK

    def body(x_ref, o_ref):
        def step(h, carry):
            o_ref[h] = x_ref[:, h, :].T
            return carry

        lax.fori_loop(0, HIST, step, 0)

    return pl.pallas_call(
        body,
        grid=(grid,),
        in_specs=[pl.BlockSpec((BBLK, HIST, DIM), lambda k: (k, 0, 0))],
        out_specs=pl.BlockSpec((HIST, DIM, BBLK), lambda k: (0, 0, k)),
        out_shape=jax.ShapeDtypeStruct((HIST, DIM, BATCH), jnp.float32),
        compiler_params=pltpu.CompilerParams(vmem_limit_bytes=100 * 2**20),
    )(arr3)


def _sc_gather(idx, table):
    mesh = plsc.VectorSubcoreMesh(core_axis_name="c", subcore_axis_name="s")

    @functools.partial(
        pl.kernel,
        mesh=mesh,
        out_type=jax.ShapeDtypeStruct((TOTAL, DIM), jnp.float32),
        scratch_types=[
            pltpu.VMEM((PER_W,), jnp.int32),
            pltpu.VMEM((CHUNK, DIM), jnp.float32),
            pltpu.VMEM((CHUNK, DIM), jnp.float32),
            pltpu.SemaphoreType.DMA,
            pltpu.SemaphoreType.DMA,
            pltpu.SemaphoreType.DMA,
            pltpu.SemaphoreType.DMA,
        ],
        compiler_params=pltpu.CompilerParams(use_tc_tiling_on_sc=False),
    )
    def k(idx_hbm, table_hbm, out_hbm, idx_v, rows0, rows1, sg0, sg1, sw0, sw1):
        wid = lax.axis_index("s") * NC + lax.axis_index("c")
        base = wid * PER_W
        pltpu.sync_copy(idx_hbm.at[pl.ds(base, PER_W)], idx_v)

        def gather_start(g, rows, sem):
            pltpu.async_copy(table_hbm.at[idx_v.at[pl.ds(g * CHUNK, CHUNK)]],
                             rows, sem)

        def gather_wait(rows, sem):
            pltpu.make_async_copy(table_hbm.at[idx_v.at[pl.ds(0, CHUNK)]],
                                  rows, sem).wait()

        def wb_start(g, rows, sem):
            pltpu.async_copy(rows, out_hbm.at[pl.ds(base + g * CHUNK, CHUNK)],
                             sem)

        def wb_wait(rows, sem):
            pltpu.make_async_copy(rows, out_hbm.at[pl.ds(0, CHUNK)], sem).wait()

        gather_start(0, rows0, sg0)

        def pair(j, carry):
            g0 = 2 * j

            @pl.when(j > 0)
            def _():
                wb_wait(rows1, sw1)

            gather_start(g0 + 1, rows1, sg1)
            gather_wait(rows0, sg0)
            wb_start(g0, rows0, sw0)

            @pl.when(j < NPAIR - 1)
            def _():
                wb_wait(rows0, sw0)
                gather_start(g0 + 2, rows0, sg0)

            gather_wait(rows1, sg1)
            wb_start(g0 + 1, rows1, sw1)
            return carry

        lax.fori_loop(0, NPAIR, pair, 0)
        wb_wait(rows0, sw0)
        wb_wait(rows1, sw1)

    return k(idx, table)


def kernel(channel_seq, W):
    table = _tc_table_transpose(W.T)
    idx = channel_seq.reshape(TOTAL).astype(jnp.int32)
    out_lin = _sc_gather(idx, table)
    outp = _tc_out_convert(out_lin.reshape(BATCH, HIST, DIM))
    return outp.transpose(2, 0, 1)


# speedup vs baseline: 1.1831x; 1.1831x over previous
import functools

import jax
import jax.numpy as jnp
from jax import lax
from jax.experimental import pallas as pl
from jax.experimental.pallas import tpu as pltpu
from jax.experimental.pallas import tpu_sc as plsc

BATCH = 4096
HIST = 200
DIM = 64
TOTAL = BATCH * HIST
NUM_EMB = 1000000

_INFO = plsc.get_sparse_core_info()
NC = _INFO.num_cores
NS = _INFO.num_subcores
NW = NC * NS
B_W = BATCH // NW
NBUF = 4
NOUTER = HIST // NBUF

TBLK = 2048


def _eye(n):
    r = lax.broadcasted_iota(jnp.int32, (n, n), 0)
    c = lax.broadcasted_iota(jnp.int32, (n, n), 1)
    return (r == c).astype(jnp.float32)


def _tc_table_transpose(wt):
    grid = (NUM_EMB + TBLK - 1) // TBLK

    def body(x_ref, o_ref):
        ident = _eye(DIM)
        x = x_ref[...]
        o_ref[...] = lax.dot_general(
            x, ident, (((0,), (0,)), ((), ())),
            preferred_element_type=jnp.float32)

    return pl.pallas_call(
        body,
        grid=(grid,),
        in_specs=[pl.BlockSpec((DIM, TBLK), lambda k: (0, k))],
        out_specs=pl.BlockSpec((TBLK, DIM), lambda k: (k, 0)),
        out_shape=jax.ShapeDtypeStruct((NUM_EMB, DIM), jnp.float32),
    )(wt)


def _tc_out_convert(arr3):
    def body(x_ref, o_ref):
        ident = _eye(DIM)
        x = x_ref[0]
        o_ref[0] = lax.dot_general(
            ident, x, (((1,), (1,)), ((), ())),
            preferred_element_type=jnp.float32)

    return pl.pallas_call(
        body,
        grid=(HIST,),
        in_specs=[pl.BlockSpec((1, BATCH, DIM), lambda h: (h, 0, 0))],
        out_specs=pl.BlockSpec((1, DIM, BATCH), lambda h: (h, 0, 0)),
        out_shape=jax.ShapeDtypeStruct((HIST, DIM, BATCH), jnp.float32),
        compiler_params=pltpu.CompilerParams(vmem_limit_bytes=100 * 2**20),
    )(arr3)


def _sc_gather(idx2d, table):
    mesh = plsc.VectorSubcoreMesh(core_axis_name="c", subcore_axis_name="s")

    @functools.partial(
        pl.kernel,
        mesh=mesh,
        out_type=jax.ShapeDtypeStruct((TOTAL, DIM), jnp.float32),
        scratch_types=[
            pltpu.VMEM((HIST, B_W), jnp.int32),
            pltpu.VMEM((NBUF, B_W, DIM), jnp.float32),
            pltpu.SemaphoreType.DMA,
            pltpu.SemaphoreType.DMA,
            pltpu.SemaphoreType.DMA,
            pltpu.SemaphoreType.DMA,
            pltpu.SemaphoreType.DMA,
            pltpu.SemaphoreType.DMA,
            pltpu.SemaphoreType.DMA,
            pltpu.SemaphoreType.DMA,
        ],
        compiler_params=pltpu.CompilerParams(use_tc_tiling_on_sc=False),
    )
    def k(idx_hbm, table_hbm, out_hbm, idx_v, rows, *sems):
        sg = sems[:NBUF]
        sw = sems[NBUF:]
        wid = lax.axis_index("s") * NC + lax.axis_index("c")
        b0 = wid * B_W
        pltpu.sync_copy(idx_hbm.at[:, pl.ds(b0, B_W)], idx_v)

        def gather_start(h, s):
            pltpu.async_copy(table_hbm.at[idx_v.at[h]], rows.at[s], sg[s])

        def gather_wait(s):
            pltpu.make_async_copy(table_hbm.at[idx_v.at[0]], rows.at[s],
                                  sg[s]).wait()

        def wb_start(h, s):
            pltpu.async_copy(rows.at[s],
                             out_hbm.at[pl.ds(h * BATCH + b0, B_W)], sw[s])

        def wb_wait(s):
            pltpu.make_async_copy(rows.at[s], out_hbm.at[pl.ds(0, B_W)],
                                  sw[s]).wait()

        for s in range(NBUF):
            gather_start(s, s)

        def outer(jo, carry):
            for s in range(NBUF):
                h = jo * NBUF + s
                gather_wait(s)
                wb_start(h, s)
                nh = h + NBUF

                @pl.when(nh < HIST)
                def _():
                    wb_wait(s)
                    gather_start(nh, s)

            return carry

        lax.fori_loop(0, NOUTER, outer, 0)
        for s in range(NBUF):
            wb_wait(s)

    return k(idx2d, table)


def kernel(channel_seq, W):
    table = _tc_table_transpose(W.T)
    out_mid = _sc_gather(channel_seq.T, table)
    outp = _tc_out_convert(out_mid.reshape(HIST, BATCH, DIM))
    return outp.transpose(2, 0, 1)


# speedup vs baseline: 1.4433x; 1.2200x over previous
import functools

import jax
import jax.numpy as jnp
from jax import lax
from jax.experimental import pallas as pl
from jax.experimental.pallas import tpu as pltpu
from jax.experimental.pallas import tpu_sc as plsc

BATCH = 4096
HIST = 200
DIM = 64
TOTAL = BATCH * HIST
NUM_EMB = 1000000

_INFO = plsc.get_sparse_core_info()
NC = _INFO.num_cores
NS = _INFO.num_subcores
NW = NC * NS
B_W = BATCH // NW
NBUF = 4
NOUTER = HIST // NBUF

TBLK = 8192
HBLK = 4


def _eye(n):
    r = lax.broadcasted_iota(jnp.int32, (n, n), 0)
    c = lax.broadcasted_iota(jnp.int32, (n, n), 1)
    return (r == c).astype(jnp.float32)


def _tc_table_transpose(wt):
    grid = (NUM_EMB + TBLK - 1) // TBLK

    def body(x_ref, o_ref):
        ident = _eye(DIM)
        x = x_ref[...]
        o_ref[...] = lax.dot_general(
            x, ident, (((0,), (0,)), ((), ())),
            preferred_element_type=jnp.float32)

    return pl.pallas_call(
        body,
        grid=(grid,),
        in_specs=[pl.BlockSpec((DIM, TBLK), lambda k: (0, k))],
        out_specs=pl.BlockSpec((TBLK, DIM), lambda k: (k, 0)),
        out_shape=jax.ShapeDtypeStruct((NUM_EMB, DIM), jnp.float32),
        compiler_params=pltpu.CompilerParams(vmem_limit_bytes=100 * 2**20),
    )(wt)


def _tc_out_convert(arr3):
    def body(x_ref, o_ref):
        ident = _eye(DIM)
        for i in range(HBLK):
            x = x_ref[i]
            o_ref[i] = lax.dot_general(
                ident, x, (((1,), (1,)), ((), ())),
                preferred_element_type=jnp.float32)

    return pl.pallas_call(
        body,
        grid=(HIST // HBLK,),
        in_specs=[pl.BlockSpec((HBLK, BATCH, DIM), lambda h: (h, 0, 0))],
        out_specs=pl.BlockSpec((HBLK, DIM, BATCH), lambda h: (h, 0, 0)),
        out_shape=jax.ShapeDtypeStruct((HIST, DIM, BATCH), jnp.float32),
        compiler_params=pltpu.CompilerParams(vmem_limit_bytes=100 * 2**20),
    )(arr3)


def _sc_gather(idx2d, table):
    mesh = plsc.VectorSubcoreMesh(core_axis_name="c", subcore_axis_name="s")

    @functools.partial(
        pl.kernel,
        mesh=mesh,
        out_type=jax.ShapeDtypeStruct((TOTAL, DIM), jnp.float32),
        scratch_types=[
            pltpu.VMEM((HIST, B_W), jnp.int32),
            pltpu.VMEM((NBUF, B_W, DIM), jnp.float32),
            pltpu.SemaphoreType.DMA,
            pltpu.SemaphoreType.DMA,
            pltpu.SemaphoreType.DMA,
            pltpu.SemaphoreType.DMA,
            pltpu.SemaphoreType.DMA,
            pltpu.SemaphoreType.DMA,
            pltpu.SemaphoreType.DMA,
            pltpu.SemaphoreType.DMA,
        ],
        compiler_params=pltpu.CompilerParams(use_tc_tiling_on_sc=False),
    )
    def k(idx_hbm, table_hbm, out_hbm, idx_v, rows, *sems):
        sg = sems[:NBUF]
        sw = sems[NBUF:]
        wid = lax.axis_index("s") * NC + lax.axis_index("c")
        b0 = wid * B_W
        pltpu.sync_copy(idx_hbm.at[:, pl.ds(b0, B_W)], idx_v)

        def gather_start(h, s):
            pltpu.async_copy(table_hbm.at[idx_v.at[h]], rows.at[s], sg[s])

        def gather_wait(s):
            pltpu.make_async_copy(table_hbm.at[idx_v.at[0]], rows.at[s],
                                  sg[s]).wait()

        def wb_start(h, s):
            pltpu.async_copy(rows.at[s],
                             out_hbm.at[pl.ds(h * BATCH + b0, B_W)], sw[s])

        def wb_wait(s):
            pltpu.make_async_copy(rows.at[s], out_hbm.at[pl.ds(0, B_W)],
                                  sw[s]).wait()

        for s in range(NBUF):
            gather_start(s, s)

        def outer(jo, carry):
            for s in range(NBUF):
                h = jo * NBUF + s
                gather_wait(s)
                wb_start(h, s)
                nh = h + NBUF

                @pl.when(nh < HIST)
                def _():
                    wb_wait(s)
                    gather_start(nh, s)

            return carry

        lax.fori_loop(0, NOUTER, outer, 0)
        for s in range(NBUF):
            wb_wait(s)

    return k(idx2d, table)


def kernel(channel_seq, W):
    table = _tc_table_transpose(W.T)
    out_mid = _sc_gather(channel_seq.T, table)
    outp = _tc_out_convert(out_mid.reshape(HIST, BATCH, DIM))
    return outp.transpose(2, 0, 1)


# speedup vs baseline: 1.4750x; 1.0219x over previous
import functools

import jax
import jax.numpy as jnp
from jax import lax
from jax.experimental import pallas as pl
from jax.experimental.pallas import tpu as pltpu
from jax.experimental.pallas import tpu_sc as plsc

BATCH = 4096
HIST = 200
DIM = 64
TOTAL = BATCH * HIST
NUM_EMB = 1000000

_INFO = plsc.get_sparse_core_info()
NC = _INFO.num_cores
NS = _INFO.num_subcores
NW = NC * NS
B_W = BATCH // NW
NBUF = 4
NOUTER = HIST // NBUF

TBLK = 16384
HBLK = 8


def _eye(n):
    r = lax.broadcasted_iota(jnp.int32, (n, n), 0)
    c = lax.broadcasted_iota(jnp.int32, (n, n), 1)
    return (r == c).astype(jnp.float32)


def _tc_table_transpose(wt):
    grid = (NUM_EMB + TBLK - 1) // TBLK

    def body(x_ref, o_ref):
        ident = _eye(DIM)
        x = x_ref[...]
        o_ref[...] = lax.dot_general(
            x, ident, (((0,), (0,)), ((), ())),
            preferred_element_type=jnp.float32)

    return pl.pallas_call(
        body,
        grid=(grid,),
        in_specs=[pl.BlockSpec((DIM, TBLK), lambda k: (0, k))],
        out_specs=pl.BlockSpec((TBLK, DIM), lambda k: (k, 0)),
        out_shape=jax.ShapeDtypeStruct((NUM_EMB, DIM), jnp.float32),
        compiler_params=pltpu.CompilerParams(vmem_limit_bytes=100 * 2**20),
    )(wt)


def _tc_out_convert(arr3):
    def body(x_ref, o_ref):
        ident = _eye(DIM)
        for i in range(HBLK):
            x = x_ref[i]
            o_ref[i] = lax.dot_general(
                ident, x, (((1,), (1,)), ((), ())),
                preferred_element_type=jnp.float32)

    return pl.pallas_call(
        body,
        grid=(HIST // HBLK,),
        in_specs=[pl.BlockSpec((HBLK, BATCH, DIM), lambda h: (h, 0, 0))],
        out_specs=pl.BlockSpec((HBLK, DIM, BATCH), lambda h: (h, 0, 0)),
        out_shape=jax.ShapeDtypeStruct((HIST, DIM, BATCH), jnp.float32),
        compiler_params=pltpu.CompilerParams(vmem_limit_bytes=100 * 2**20),
    )(arr3)


def _sc_gather(idx2d, table):
    mesh = plsc.VectorSubcoreMesh(core_axis_name="c", subcore_axis_name="s")

    @functools.partial(
        pl.kernel,
        mesh=mesh,
        out_type=jax.ShapeDtypeStruct((TOTAL, DIM), jnp.float32),
        scratch_types=[
            pltpu.VMEM((HIST, B_W), jnp.int32),
            pltpu.VMEM((NBUF, B_W, DIM), jnp.float32),
            pltpu.SemaphoreType.DMA,
            pltpu.SemaphoreType.DMA,
            pltpu.SemaphoreType.DMA,
            pltpu.SemaphoreType.DMA,
            pltpu.SemaphoreType.DMA,
            pltpu.SemaphoreType.DMA,
            pltpu.SemaphoreType.DMA,
            pltpu.SemaphoreType.DMA,
        ],
        compiler_params=pltpu.CompilerParams(use_tc_tiling_on_sc=False),
    )
    def k(idx_hbm, table_hbm, out_hbm, idx_v, rows, *sems):
        sg = sems[:NBUF]
        sw = sems[NBUF:]
        wid = lax.axis_index("s") * NC + lax.axis_index("c")
        b0 = wid * B_W
        pltpu.sync_copy(idx_hbm.at[:, pl.ds(b0, B_W)], idx_v)

        def gather_start(h, s):
            pltpu.async_copy(table_hbm.at[idx_v.at[h]], rows.at[s], sg[s])

        def gather_wait(s):
            pltpu.make_async_copy(table_hbm.at[idx_v.at[0]], rows.at[s],
                                  sg[s]).wait()

        def wb_start(h, s):
            pltpu.async_copy(rows.at[s],
                             out_hbm.at[pl.ds(h * BATCH + b0, B_W)], sw[s])

        def wb_wait(s):
            pltpu.make_async_copy(rows.at[s], out_hbm.at[pl.ds(0, B_W)],
                                  sw[s]).wait()

        for s in range(NBUF):
            gather_start(s, s)

        def outer(jo, carry):
            for s in range(NBUF):
                h = jo * NBUF + s
                gather_wait(s)
                wb_start(h, s)
                nh = h + NBUF

                @pl.when(nh < HIST)
                def _():
                    wb_wait(s)
                    gather_start(nh, s)

            return carry

        lax.fori_loop(0, NOUTER, outer, 0)
        for s in range(NBUF):
            wb_wait(s)

    return k(idx2d, table)


def kernel(channel_seq, W):
    table = _tc_table_transpose(W.T)
    out_mid = _sc_gather(channel_seq.T, table)
    outp = _tc_out_convert(out_mid.reshape(HIST, BATCH, DIM))
    return outp.transpose(2, 0, 1)


# speedup vs baseline: 2.3068x; 1.5639x over previous
import functools

import jax
import jax.numpy as jnp
from jax import lax
from jax.experimental import pallas as pl
from jax.experimental.pallas import tpu as pltpu
from jax.experimental.pallas import tpu_sc as plsc

BATCH = 4096
HIST = 200
DIM = 64
TOTAL = BATCH * HIST
NUM_EMB = 1000000

_INFO = plsc.get_sparse_core_info()
NC = _INFO.num_cores
NS = _INFO.num_subcores
NW = NC * NS
B_W = BATCH // NW
NBUF = 4
NOUTER = HIST // NBUF

TBLK = 16384
HBLK = 8


def _eye(n):
    r = lax.broadcasted_iota(jnp.int32, (n, n), 0)
    c = lax.broadcasted_iota(jnp.int32, (n, n), 1)
    return (r == c).astype(jnp.float32)


SPLIT = 31 * TBLK


def _tc_table_transpose(wt):
    grid = SPLIT // TBLK

    def body(xa_ref, xb_ref, o_ref):
        ident = _eye(2 * DIM)
        x2 = jnp.concatenate([xa_ref[...], xb_ref[...]], axis=0)
        o_ref[...] = lax.dot_general(
            x2, ident, (((0,), (0,)), ((), ())),
            preferred_element_type=jnp.float32)

    return pl.pallas_call(
        body,
        grid=(grid,),
        in_specs=[
            pl.BlockSpec((DIM, TBLK), lambda k: (0, k)),
            pl.BlockSpec((DIM, TBLK), lambda k: (0, k + 31)),
        ],
        out_specs=pl.BlockSpec((TBLK, 2 * DIM), lambda k: (k, 0)),
        out_shape=jax.ShapeDtypeStruct((SPLIT, 2 * DIM), jnp.float32),
        compiler_params=pltpu.CompilerParams(vmem_limit_bytes=100 * 2**20),
    )(wt, wt)


def _tc_out_convert(arr3):
    def body(x_ref, o_ref):
        ident = _eye(DIM)
        for i in range(HBLK):
            x = x_ref[i]
            o_ref[i] = lax.dot_general(
                ident, x, (((1,), (1,)), ((), ())),
                preferred_element_type=jnp.float32)

    return pl.pallas_call(
        body,
        grid=(HIST // HBLK,),
        in_specs=[pl.BlockSpec((HBLK, BATCH, DIM), lambda h: (h, 0, 0))],
        out_specs=pl.BlockSpec((HBLK, DIM, BATCH), lambda h: (h, 0, 0)),
        out_shape=jax.ShapeDtypeStruct((HIST, DIM, BATCH), jnp.float32),
        compiler_params=pltpu.CompilerParams(vmem_limit_bytes=100 * 2**20),
    )(arr3)


def _sc_gather(idx2d, table):
    mesh = plsc.VectorSubcoreMesh(core_axis_name="c", subcore_axis_name="s")

    @functools.partial(
        pl.kernel,
        mesh=mesh,
        out_type=jax.ShapeDtypeStruct((TOTAL, DIM), jnp.float32),
        scratch_types=[
            pltpu.VMEM((HIST, B_W), jnp.int32),
            pltpu.VMEM((NBUF, B_W, DIM), jnp.float32),
            pltpu.SemaphoreType.DMA,
            pltpu.SemaphoreType.DMA,
            pltpu.SemaphoreType.DMA,
            pltpu.SemaphoreType.DMA,
            pltpu.SemaphoreType.DMA,
            pltpu.SemaphoreType.DMA,
            pltpu.SemaphoreType.DMA,
            pltpu.SemaphoreType.DMA,
        ],
        compiler_params=pltpu.CompilerParams(use_tc_tiling_on_sc=False),
    )
    def k(idx_hbm, table_hbm, out_hbm, idx_v, rows, *sems):
        sg = sems[:NBUF]
        sw = sems[NBUF:]
        wid = lax.axis_index("s") * NC + lax.axis_index("c")
        b0 = wid * B_W
        pltpu.sync_copy(idx_hbm.at[:, pl.ds(b0, B_W)], idx_v)

        def remap(h, carry):
            for g in range(B_W // 16):
                v = idx_v[h, pl.ds(g * 16, 16)]
                idx_v[h, pl.ds(g * 16, 16)] = jnp.where(
                    v < SPLIT, 2 * v, 2 * v - (2 * SPLIT - 1))
            return carry

        lax.fori_loop(0, HIST, remap, 0)

        def gather_start(h, s):
            pltpu.async_copy(table_hbm.at[idx_v.at[h]], rows.at[s], sg[s])

        def gather_wait(s):
            pltpu.make_async_copy(table_hbm.at[idx_v.at[0]], rows.at[s],
                                  sg[s]).wait()

        def wb_start(h, s):
            pltpu.async_copy(rows.at[s],
                             out_hbm.at[pl.ds(h * BATCH + b0, B_W)], sw[s])

        def wb_wait(s):
            pltpu.make_async_copy(rows.at[s], out_hbm.at[pl.ds(0, B_W)],
                                  sw[s]).wait()

        for s in range(NBUF):
            gather_start(s, s)

        def outer(jo, carry):
            for s in range(NBUF):
                h = jo * NBUF + s
                gather_wait(s)
                wb_start(h, s)
                nh = h + NBUF

                @pl.when(nh < HIST)
                def _():
                    wb_wait(s)
                    gather_start(nh, s)

            return carry

        lax.fori_loop(0, NOUTER, outer, 0)
        for s in range(NBUF):
            wb_wait(s)

    return k(idx2d, table)


def kernel(channel_seq, W):
    table = _tc_table_transpose(W.T).reshape(2 * SPLIT, DIM)
    out_mid = _sc_gather(channel_seq.T, table)
    outp = _tc_out_convert(out_mid.reshape(HIST, BATCH, DIM))
    return outp.transpose(2, 0, 1)


# speedup vs baseline: 4.1917x; 1.8171x over previous
import functools

import jax
import jax.numpy as jnp
from jax import lax
from jax.experimental import pallas as pl
from jax.experimental.pallas import tpu as pltpu
from jax.experimental.pallas import tpu_sc as plsc

BATCH = 4096
HIST = 200
DIM = 64
TOTAL = BATCH * HIST
NUM_EMB = 1000000

_INFO = plsc.get_sparse_core_info()
NC = _INFO.num_cores
NS = _INFO.num_subcores
NW = NC * NS
B_W = BATCH // NW
NBUF = 4
NOUTER = HIST // NBUF

TBLK = 16384
HBLK = 8


def _eye(n):
    r = lax.broadcasted_iota(jnp.int32, (n, n), 0)
    c = lax.broadcasted_iota(jnp.int32, (n, n), 1)
    return (r == c).astype(jnp.float32)


SPLIT = 31 * TBLK


def _tc_table_transpose(wt):
    grid = SPLIT // TBLK

    def body(xa_ref, xb_ref, o_ref):
        ident = _eye(2 * DIM)
        x2 = jnp.concatenate([xa_ref[...], xb_ref[...]], axis=0)
        o_ref[...] = lax.dot_general(
            x2, ident, (((0,), (0,)), ((), ())),
            preferred_element_type=jnp.float32)

    return pl.pallas_call(
        body,
        grid=(grid,),
        in_specs=[
            pl.BlockSpec((DIM, TBLK), lambda k: (0, k)),
            pl.BlockSpec((DIM, TBLK), lambda k: (0, k + 31)),
        ],
        out_specs=pl.BlockSpec((TBLK, 2 * DIM), lambda k: (k, 0)),
        out_shape=jax.ShapeDtypeStruct((SPLIT, 2 * DIM), jnp.float32),
        compiler_params=pltpu.CompilerParams(vmem_limit_bytes=100 * 2**20),
    )(wt, wt)


HB = BATCH // 2


def _tc_out_convert(arr3):
    def body(x_ref, o_ref):
        ident = _eye(2 * DIM)
        for i in range(HBLK):
            x2 = x_ref[i]
            y = lax.dot_general(
                ident, x2, (((1,), (1,)), ((), ())),
                preferred_element_type=jnp.float32)
            o_ref[i] = jnp.concatenate([y[:DIM], y[DIM:]], axis=1)

    return pl.pallas_call(
        body,
        grid=(HIST // HBLK,),
        in_specs=[pl.BlockSpec((HBLK, HB, 2 * DIM), lambda h: (h, 0, 0))],
        out_specs=pl.BlockSpec((HBLK, DIM, BATCH), lambda h: (h, 0, 0)),
        out_shape=jax.ShapeDtypeStruct((HIST, DIM, BATCH), jnp.float32),
        compiler_params=pltpu.CompilerParams(vmem_limit_bytes=100 * 2**20),
    )(arr3)


def _sc_gather(idx2d, table):
    mesh = plsc.VectorSubcoreMesh(core_axis_name="c", subcore_axis_name="s")

    @functools.partial(
        pl.kernel,
        mesh=mesh,
        out_type=jax.ShapeDtypeStruct((HIST * (BATCH // 2), 2, DIM),
                                      jnp.float32),
        scratch_types=[
            pltpu.VMEM((HIST, B_W), jnp.int32),
            pltpu.VMEM((NBUF, B_W, DIM), jnp.float32),
            pltpu.SemaphoreType.DMA,
            pltpu.SemaphoreType.DMA,
            pltpu.SemaphoreType.DMA,
            pltpu.SemaphoreType.DMA,
            pltpu.SemaphoreType.DMA,
            pltpu.SemaphoreType.DMA,
            pltpu.SemaphoreType.DMA,
            pltpu.SemaphoreType.DMA,
        ],
        compiler_params=pltpu.CompilerParams(use_tc_tiling_on_sc=False),
    )
    def k(idx_hbm, table_hbm, out_hbm, idx_v, rows, *sems):
        sg = sems[:NBUF]
        sw = sems[NBUF:]
        wid = lax.axis_index("s") * NC + lax.axis_index("c")
        b0 = wid * B_W
        pltpu.sync_copy(idx_hbm.at[:, pl.ds(b0, B_W)], idx_v)

        def remap(h, carry):
            for g in range(B_W // 16):
                v = idx_v[h, pl.ds(g * 16, 16)]
                idx_v[h, pl.ds(g * 16, 16)] = jnp.where(
                    v < SPLIT, 2 * v, 2 * v - (2 * SPLIT - 1))
            return carry

        lax.fori_loop(0, HIST, remap, 0)

        def gather_start(h, s):
            pltpu.async_copy(table_hbm.at[idx_v.at[h]], rows.at[s], sg[s])

        def gather_wait(s):
            pltpu.make_async_copy(table_hbm.at[idx_v.at[0]], rows.at[s],
                                  sg[s]).wait()

        q = b0 // (BATCH // 2)
        p_off = b0 % (BATCH // 2)

        def wb_start(h, s):
            pltpu.async_copy(
                rows.at[s],
                out_hbm.at[pl.ds(h * (BATCH // 2) + p_off, B_W), q], sw[s])

        def wb_wait(s):
            pltpu.make_async_copy(rows.at[s], out_hbm.at[pl.ds(0, B_W), q],
                                  sw[s]).wait()

        for s in range(NBUF):
            gather_start(s, s)

        def outer(jo, carry):
            for s in range(NBUF):
                h = jo * NBUF + s
                gather_wait(s)
                wb_start(h, s)
                nh = h + NBUF

                @pl.when(nh < HIST)
                def _():
                    wb_wait(s)
                    gather_start(nh, s)

            return carry

        lax.fori_loop(0, NOUTER, outer, 0)
        for s in range(NBUF):
            wb_wait(s)

    return k(idx2d, table)


def kernel(channel_seq, W):
    table = _tc_table_transpose(W.T).reshape(2 * SPLIT, DIM)
    out_mid = _sc_gather(channel_seq.T, table)
    outp = _tc_out_convert(out_mid.reshape(HIST, BATCH // 2, 2 * DIM))
    return outp.transpose(2, 0, 1)
